# Initial kernel scaffold; baseline (speedup 1.0000x reference)
import jax
import jax.numpy as jnp
from jax.experimental import pallas as pl


def kernel(cat_input_ids, cat_null_ids, cat_token_type, num_input_ids, num_null_ids, num_token_type, text_input_ids, text_token_type, cat_table, cat_pos_emb, num_emb_param, num_pos_emb, text_table, tt_table, null_table, pe_table):
    raise NotImplementedError("write your pallas kernel here")


# speedup vs baseline: 3.9386x; 3.9386x over previous
import functools

import jax
import jax.numpy as jnp
from jax import lax
from jax.experimental import pallas as pl
from jax.experimental.pallas import tpu as pltpu
import jax.experimental.pallas.tpu_sc as plsc

_B = 4096
_NCAT = 26
_NNUM = 26
_NTEXT = 50
_D = 128
_P = _NCAT + _NNUM + _NTEXT
_W = 32
_CK = 128
_G = 2

_CAT_CH = _B * _NCAT // (_W * _CK)
_NUM_CH = _B * _NNUM // (_W * _CK)
_TXT_CH = _B * _NTEXT // (_W * _CK)
_NUM_SLAB = _NUM_CH * _CK


def _sc_body(cat_tbl, text_tbl, cat_add, num_add, text_add, param, x_t,
             cgi, cai, coi, nai, noi, tgi, tai, toi,
             out,
             cg_v, ca_v, co_v, na_v, no_v, tg_v, ta_v, to_v,
             param_v, x_v, buf, sem_a, sem_b, sem_c):
    cid = lax.axis_index("c")
    sid = lax.axis_index("s")
    w = sid * 2 + cid

    pltpu.sync_copy(cgi.at[w], cg_v)
    pltpu.sync_copy(cai.at[w], ca_v)
    pltpu.sync_copy(coi.at[w], co_v)
    pltpu.sync_copy(nai.at[w], na_v)
    pltpu.sync_copy(noi.at[w], no_v)
    pltpu.sync_copy(tgi.at[w], tg_v)
    pltpu.sync_copy(tai.at[w], ta_v)
    pltpu.sync_copy(toi.at[w], to_v)
    pltpu.sync_copy(param, param_v)
    pltpu.sync_copy(x_t.at[w], x_v)

    def gather_section(n_sup, tbl, add_tbl, gi_v, ai_v, oi_v):
        def sup_body(g, carry):
            d = [pltpu.async_copy(add_tbl.at[ai_v.at[g * _G + k]],
                                  buf.at[pl.ds(k * _CK, _CK)], sem_a)
                 for k in range(_G)]
            for x in d:
                x.wait()
            d = [pltpu.async_copy(tbl.at[gi_v.at[g * _G + k]],
                                  buf.at[pl.ds(k * _CK, _CK)], sem_b, add=True)
                 for k in range(_G)]
            for x in d:
                x.wait()
            d = [pltpu.async_copy(buf.at[pl.ds(k * _CK, _CK)],
                                  out.at[oi_v.at[g * _G + k]], sem_c)
                 for k in range(_G)]
            for x in d:
                x.wait()
            return carry
        lax.fori_loop(0, n_sup, sup_body, 0)

    gather_section(_CAT_CH // _G, cat_tbl, cat_add, cg_v, ca_v, co_v)

    def num_sup(g, carry):
        d = [pltpu.async_copy(num_add.at[na_v.at[g * _G + k]],
                              buf.at[pl.ds(k * _CK, _CK)], sem_a)
             for k in range(_G)]
        for x in d:
            x.wait()
        for k in range(_G):
            cc = g * _G + k
            jrow = (w * _NUM_SLAB + cc * _CK) // _B
            pv = [param_v[jrow, pl.ds(t * 16, 16)] for t in range(8)]

            def row_body(q, inner):
                xv = x_v[pl.ds(cc * _CK + q * 16, 16)]
                for r in range(16):
                    xs = xv[r]
                    row = k * _CK + q * 16 + r
                    for t in range(8):
                        buf[row, pl.ds(t * 16, 16)] = (
                            buf[row, pl.ds(t * 16, 16)] + xs * pv[t])
                return inner
            lax.fori_loop(0, _CK // 16, row_body, 0)
        d = [pltpu.async_copy(buf.at[pl.ds(k * _CK, _CK)],
                              out.at[no_v.at[g * _G + k]], sem_c)
             for k in range(_G)]
        for x in d:
            x.wait()
        return carry
    lax.fori_loop(0, _NUM_CH // _G, num_sup, 0)

    gather_section(_TXT_CH // _G, text_tbl, text_add, tg_v, ta_v, to_v)


@functools.partial(
    pl.kernel,
    out_type=jax.ShapeDtypeStruct((_B * _P, _D), jnp.float32),
    mesh=plsc.VectorSubcoreMesh(core_axis_name="c", subcore_axis_name="s",
                                num_cores=2, num_subcores=16),
    scratch_types=[
        pltpu.VMEM((_CAT_CH, _CK), jnp.int32),
        pltpu.VMEM((_CAT_CH, _CK), jnp.int32),
        pltpu.VMEM((_CAT_CH, _CK), jnp.int32),
        pltpu.VMEM((_NUM_CH, _CK), jnp.int32),
        pltpu.VMEM((_NUM_CH, _CK), jnp.int32),
        pltpu.VMEM((_TXT_CH, _CK), jnp.int32),
        pltpu.VMEM((_TXT_CH, _CK), jnp.int32),
        pltpu.VMEM((_TXT_CH, _CK), jnp.int32),
        pltpu.VMEM((_NNUM, _D), jnp.float32),
        pltpu.VMEM((_NUM_SLAB,), jnp.float32),
        pltpu.VMEM((_G * _CK, _D), jnp.float32),
        pltpu.SemaphoreType.DMA,
        pltpu.SemaphoreType.DMA,
        pltpu.SemaphoreType.DMA,
    ],
)
def _sc_kernel(*args):
    _sc_body(*args)


def kernel(cat_input_ids, cat_null_ids, cat_token_type, num_input_ids,
           num_null_ids, num_token_type, text_input_ids, text_token_type,
           cat_table, cat_pos_emb, num_emb_param, num_pos_emb,
           text_table, tt_table, null_table, pe_table):
    i32 = jnp.int32
    cat_add = (cat_pos_emb[:, None, None, :] + tt_table[None, :, None, :]
               + null_table[None, None, :, :]).reshape(_NCAT * 6, _D)
    num_add = (num_pos_emb[:, None, None, :] + tt_table[None, :, None, :]
               + null_table[None, None, :, :]).reshape(_NNUM * 6, _D)
    text_add = (pe_table[:, None, :] + tt_table[None, :, :]).reshape(
        _NTEXT * 2, _D)

    b = jnp.arange(_B, dtype=i32)[:, None]
    jc = jnp.arange(_NCAT, dtype=i32)[None, :]
    jt = jnp.arange(_NTEXT, dtype=i32)[None, :]

    cgi = cat_input_ids.astype(i32).reshape(_W, _CAT_CH, _CK)
    cai = (jc * 6 + cat_token_type.astype(i32) * 3
           + cat_null_ids.astype(i32)).reshape(_W, _CAT_CH, _CK)
    coi = (b * _P + jc).reshape(_W, _CAT_CH, _CK)

    nai = (jc * 6 + num_token_type.astype(i32) * 3
           + num_null_ids.astype(i32)).T.reshape(_W, _NUM_CH, _CK)
    noi = (b * _P + _NCAT + jc).T.reshape(_W, _NUM_CH, _CK)
    x_t = num_input_ids.astype(jnp.float32).T.reshape(_W, _NUM_SLAB)

    tgi = text_input_ids.astype(i32).reshape(_W, _TXT_CH, _CK)
    tai = (jt * 2 + text_token_type.astype(i32)).reshape(_W, _TXT_CH, _CK)
    toi = (b * _P + _NNUM + _NCAT + jt).reshape(_W, _TXT_CH, _CK)

    out = _sc_kernel(cat_table, text_table, cat_add, num_add, text_add,
                     num_emb_param, x_t,
                     cgi, cai, coi, nai, noi, tgi, tai, toi)
    return out.reshape(_B, _P, _D)
